# baseline (speedup 1.0000x reference)
import functools

import jax
import jax.numpy as jnp
from jax import lax
from jax.experimental import pallas as pl
from jax.experimental.pallas import tpu as pltpu
from jax.experimental.pallas import tpu_sc as plsc

H = 256
W = 256
G = 8192
TS = 16
TX = W // TS
TY = H // TS
NT = TX * TY
L = 512
LCH = L // 128
KF = 512
KD = 128
NW = 32
TPW = NT // NW
ALPHA_THR = 1.0 / 255.0
TRANS_THR = 1.0e-4


_SC_MESH = plsc.VectorSubcoreMesh(
    core_axis_name="c", subcore_axis_name="s", num_cores=2, num_subcores=16
)


@functools.partial(
    pl.kernel,
    out_type=(
        jax.ShapeDtypeStruct((NW, 16), jnp.int32),
        jax.ShapeDtypeStruct((NT, L, 16), jnp.float32),
    ),
    mesh=_SC_MESH,
    scratch_types=(
        pltpu.VMEM((4, G), jnp.int32),
        pltpu.VMEM((L + 16,), jnp.int32),
        pltpu.VMEM((L, 16), jnp.float32),
        pltpu.VMEM((16,), jnp.int32),
        pltpu.SemaphoreType.DMA,
    ),
    compiler_params=pltpu.CompilerParams(
        needs_layout_passes=False, use_tc_tiling_on_sc=False
    ),
)
def _sc_bin(bbox_hbm, params_hbm, counts_hbm, binned_hbm,
            bbox_v, list_v, rows_v, cnts_v, sem):
    wid = lax.axis_index("s") * 2 + lax.axis_index("c")
    pltpu.sync_copy(bbox_hbm, bbox_v)

    zeros16 = jnp.zeros((16,), jnp.int32)
    for i in range((L + 16) // 16):
        list_v[pl.ds(i * 16, 16)] = zeros16

    lane = lax.broadcasted_iota(jnp.int32, (16,), 0)
    counts_vec = zeros16

    for k in range(TPW):
        t = wid * TPW + k
        txs = lax.rem(t, TX)
        tys = lax.div(t, TX)

        def body(j, cnt):
            tx0 = bbox_v[0, pl.ds(j * 16, 16)]
            tx1 = bbox_v[1, pl.ds(j * 16, 16)]
            ty0 = bbox_v[2, pl.ds(j * 16, 16)]
            ty1 = bbox_v[3, pl.ds(j * 16, 16)]
            hit = ((tx0 <= txs) & (txs <= tx1)
                   & (ty0 <= tys) & (tys <= ty1))
            popc = plsc.all_reduce_population_count(hit)
            npop = jnp.max(popc, axis=0)

            def append(c):
                pos = plsc.cumsum(hit.astype(jnp.int32))
                idx = jnp.minimum(c, jnp.int32(L)) + pos - 1
                plsc.store_scatter(list_v, [idx], j * 16 + lane, mask=hit)
                return c + npop

            return lax.cond(npop > 0, append, lambda c: c, cnt)

        cnt = lax.fori_loop(0, G // 16, body, jnp.int32(0))
        counts_vec = jnp.where(lane == k, cnt, counts_vec)

        for ch in range(LCH):

            def move_chunk(ch=ch):
                cp = pltpu.async_copy(
                    params_hbm.at[list_v.at[pl.ds(ch * 128, 128)]],
                    rows_v.at[pl.ds(ch * 128, 128)],
                    sem,
                )
                cp.wait()
                pltpu.sync_copy(
                    rows_v.at[pl.ds(ch * 128, 128)],
                    binned_hbm.at[t, pl.ds(ch * 128, 128)],
                )

            pl.when(cnt > ch * 128)(move_chunk)

    cnts_v[...] = counts_vec
    pltpu.sync_copy(cnts_v, counts_hbm.at[wid])


def _prefix_prod_incl(t):
    k = t.shape[0]
    s = 1
    while s < k:
        t = t * jnp.concatenate(
            [jnp.ones((s, t.shape[1]), jnp.float32), t[:-s, :]], axis=0
        )
        s *= 2
    return t


def _composite_chunk(pr, gmask, px, py, carry):
    T, r, g, b = carry
    mx = pr[:, 0:1]
    my = pr[:, 1:2]
    ca = pr[:, 2:3]
    cb = pr[:, 3:4]
    cc = pr[:, 4:5]
    cr = pr[:, 5:6]
    cg = pr[:, 6:7]
    cbl = pr[:, 7:8]
    op = pr[:, 8:9]
    dx = px - mx
    dy = py - my
    sigma = 0.5 * (ca * dx * dx + cc * dy * dy) + cb * dx * dy
    al = jnp.minimum(0.99, op * jnp.exp(-sigma))
    ok = (sigma >= 0.0) & (al >= ALPHA_THR)
    if gmask is not None:
        ok = ok & gmask
    al = jnp.where(ok, al, 0.0)
    inc = _prefix_prod_incl(1.0 - al)
    npix = px.shape[1]
    tb = T * jnp.concatenate(
        [jnp.ones((1, npix), jnp.float32), inc[:-1, :]], axis=0
    )
    wgt = jnp.where(tb > TRANS_THR, al * tb, 0.0)
    r = r + jnp.sum(wgt * cr, axis=0, keepdims=True)
    g = g + jnp.sum(wgt * cg, axis=0, keepdims=True)
    b = b + jnp.sum(wgt * cbl, axis=0, keepdims=True)
    T = T * inc[-1:, :]
    return T, r, g, b


def _tile_kernel(counts_ref, binned_ref, params_ref, out_ref):
    t = pl.program_id(0)
    cnt = counts_ref[t]
    x0 = ((t % TX) * TS).astype(jnp.float32)
    y0 = ((t // TX) * TS).astype(jnp.float32)
    lane = lax.broadcasted_iota(jnp.int32, (1, TS * TS), 1)
    px = x0 + (lane % TS).astype(jnp.float32) + 0.5
    py = y0 + (lane // TS).astype(jnp.float32) + 0.5

    T0 = jnp.ones((1, TS * TS), jnp.float32)
    z = jnp.zeros((1, TS * TS), jnp.float32)
    init = (T0, z, z, z)

    def fast_fn(_):
        nch = (cnt + (KF - 1)) // KF
        gidx = lax.broadcasted_iota(jnp.int32, (KF, 1), 0)

        def chunk(j, carry):
            pr = binned_ref[0, pl.ds(j * KF, KF), :]
            gmask = (j * KF + gidx) < cnt
            return _composite_chunk(pr, gmask, px, py, carry)

        def body(j, carry):
            return lax.cond(
                jnp.any(carry[0] > TRANS_THR), lambda c: chunk(j, c),
                lambda c: c, carry)

        return lax.fori_loop(0, nch, body, init)

    def dense_fn(_):
        def chunk(i, carry):
            pr = params_ref[pl.ds(i * KD, KD), :]
            return _composite_chunk(pr, None, px, py, carry)

        def body(i, carry):
            return lax.cond(
                jnp.any(carry[0] > TRANS_THR), lambda c: chunk(i, c),
                lambda c: c, carry)

        return lax.fori_loop(0, G // KD, body, init)

    T, r, g, b = lax.cond(cnt <= L, fast_fn, dense_fn, None)
    out_ref[0, :, :] = jnp.concatenate([r, g, b], axis=0)


def _tile_ranges(params):
    mx = params[:, 0]
    my = params[:, 1]
    a = params[:, 2]
    b = params[:, 3]
    c = params[:, 4]
    op = params[:, 8]
    lnmax = jnp.log(jnp.maximum(255.0 * op, 1e-20))
    valid = lnmax > 0.0
    lnmax = jnp.maximum(lnmax, 0.0)
    rx = jnp.sqrt(2.0 * lnmax / jnp.maximum(a - b * b / c, 1e-6)) + 0.05
    ry = jnp.sqrt(2.0 * lnmax / jnp.maximum(c - b * b / a, 1e-6)) + 0.05
    tx0 = jnp.maximum(jnp.ceil((mx - rx - (TS - 0.5)) / TS), 0.0)
    tx1 = jnp.minimum(jnp.floor((mx + rx - 0.5) / TS), TX - 1.0)
    ty0 = jnp.maximum(jnp.ceil((my - ry - (TS - 0.5)) / TS), 0.0)
    ty1 = jnp.minimum(jnp.floor((my + ry - 0.5) / TS), TY - 1.0)
    tx0 = jnp.where(valid, tx0, 1.0).astype(jnp.int32)
    tx1 = jnp.where(valid, tx1, 0.0).astype(jnp.int32)
    ty0 = jnp.where(valid, ty0, 1.0).astype(jnp.int32)
    ty1 = jnp.where(valid, ty1, 0.0).astype(jnp.int32)
    return tx0, tx1, ty0, ty1


def kernel(means2d, conics, colors, opacities, depths):
    order = jnp.argsort(lax.stop_gradient(depths))
    params = jnp.concatenate(
        [
            means2d,
            conics,
            colors,
            opacities[:, None],
            jnp.zeros((G, 7), jnp.float32),
        ],
        axis=1,
    )
    params = jnp.take(params, order, axis=0)

    tx0, tx1, ty0, ty1 = _tile_ranges(params)
    bbox = jnp.stack([tx0, tx1, ty0, ty1], axis=0)

    counts2d, binned = _sc_bin(bbox, params)
    counts = counts2d[:, :TPW].reshape(NT)

    out = pl.pallas_call(
        _tile_kernel,
        grid=(NT,),
        in_specs=[
            pl.BlockSpec(memory_space=pltpu.SMEM),
            pl.BlockSpec((1, L, 16), lambda t: (t, 0, 0)),
            pl.BlockSpec((G, 16), lambda t: (0, 0)),
        ],
        out_specs=pl.BlockSpec((1, 3, TS * TS), lambda t: (t, 0, 0)),
        out_shape=jax.ShapeDtypeStruct((NT, 3, TS * TS), jnp.float32),
        compiler_params=pltpu.CompilerParams(
            dimension_semantics=("parallel",)
        ),
    )(counts, binned, params)

    img = out.reshape(TY, TX, 3, TS, TS)
    img = jnp.transpose(img, (0, 3, 1, 4, 2)).reshape(H, W, 3)
    return img


# speedup vs baseline: 1.2363x; 1.2363x over previous
import functools

import jax
import jax.numpy as jnp
from jax import lax
from jax.experimental import pallas as pl
from jax.experimental.pallas import tpu as pltpu
from jax.experimental.pallas import tpu_sc as plsc

H = 256
W = 256
G = 8192
TS = 16
TX = W // TS
TY = H // TS
NT = TX * TY
L = 512
LCH = L // 128
KF = 256
KD = 128
NW = 32
TPW = NT // NW
ALPHA_THR = 1.0 / 255.0
TRANS_THR = 1.0e-4


_SC_MESH = plsc.VectorSubcoreMesh(
    core_axis_name="c", subcore_axis_name="s", num_cores=2, num_subcores=16
)


@functools.partial(
    pl.kernel,
    out_type=(
        jax.ShapeDtypeStruct((NW, 16), jnp.int32),
        jax.ShapeDtypeStruct((NT, L, 16), jnp.float32),
    ),
    mesh=_SC_MESH,
    scratch_types=(
        pltpu.VMEM((4, G), jnp.int32),
        pltpu.VMEM((L + 16,), jnp.int32),
        pltpu.VMEM((L, 16), jnp.float32),
        pltpu.VMEM((16,), jnp.int32),
        pltpu.SemaphoreType.DMA,
    ),
    compiler_params=pltpu.CompilerParams(
        needs_layout_passes=False, use_tc_tiling_on_sc=False
    ),
)
def _sc_bin(bbox_hbm, params_hbm, counts_hbm, binned_hbm,
            bbox_v, list_v, rows_v, cnts_v, sem):
    wid = lax.axis_index("s") * 2 + lax.axis_index("c")
    pltpu.sync_copy(bbox_hbm, bbox_v)

    zeros16 = jnp.zeros((16,), jnp.int32)
    for i in range((L + 16) // 16):
        list_v[pl.ds(i * 16, 16)] = zeros16

    lane = lax.broadcasted_iota(jnp.int32, (16,), 0)
    counts_vec = zeros16

    for k in range(TPW):
        t = wid * TPW + k
        txs = lax.rem(t, TX)
        tys = lax.div(t, TX)

        def body(j, cnt):
            tx0 = bbox_v[0, pl.ds(j * 16, 16)]
            tx1 = bbox_v[1, pl.ds(j * 16, 16)]
            ty0 = bbox_v[2, pl.ds(j * 16, 16)]
            ty1 = bbox_v[3, pl.ds(j * 16, 16)]
            hit = ((tx0 <= txs) & (txs <= tx1)
                   & (ty0 <= tys) & (tys <= ty1))
            popc = plsc.all_reduce_population_count(hit)
            npop = jnp.max(popc, axis=0)

            def append(c):
                pos = plsc.cumsum(hit.astype(jnp.int32))
                idx = jnp.minimum(c, jnp.int32(L)) + pos - 1
                plsc.store_scatter(list_v, [idx], j * 16 + lane, mask=hit)
                return c + npop

            return lax.cond(npop > 0, append, lambda c: c, cnt)

        cnt = lax.fori_loop(0, G // 16, body, jnp.int32(0))
        counts_vec = jnp.where(lane == k, cnt, counts_vec)

        for ch in range(LCH):

            def move_chunk(ch=ch):
                cp = pltpu.async_copy(
                    params_hbm.at[list_v.at[pl.ds(ch * 128, 128)]],
                    rows_v.at[pl.ds(ch * 128, 128)],
                    sem,
                )
                cp.wait()
                pltpu.sync_copy(
                    rows_v.at[pl.ds(ch * 128, 128)],
                    binned_hbm.at[t, pl.ds(ch * 128, 128)],
                )

            pl.when(cnt > ch * 128)(move_chunk)

    cnts_v[...] = counts_vec
    pltpu.sync_copy(cnts_v, counts_hbm.at[wid])


def _prefix_prod_incl(t):
    k = t.shape[0]
    s = 1
    while s < k:
        t = t * jnp.concatenate(
            [jnp.ones((s, t.shape[1]), jnp.float32), t[:-s, :]], axis=0
        )
        s *= 2
    return t


def _composite_chunk(pr, gmask, px, py, carry):
    T, r, g, b = carry
    mx = pr[:, 0:1]
    my = pr[:, 1:2]
    ca = pr[:, 2:3]
    cb = pr[:, 3:4]
    cc = pr[:, 4:5]
    cr = pr[:, 5:6]
    cg = pr[:, 6:7]
    cbl = pr[:, 7:8]
    op = pr[:, 8:9]
    dx = px - mx
    dy = py - my
    sigma = 0.5 * (ca * dx * dx + cc * dy * dy) + cb * dx * dy
    al = jnp.minimum(0.99, op * jnp.exp(-sigma))
    ok = (sigma >= 0.0) & (al >= ALPHA_THR)
    if gmask is not None:
        ok = ok & gmask
    al = jnp.where(ok, al, 0.0)
    inc = _prefix_prod_incl(1.0 - al)
    npix = px.shape[1]
    tb = T * jnp.concatenate(
        [jnp.ones((1, npix), jnp.float32), inc[:-1, :]], axis=0
    )
    wgt = jnp.where(tb > TRANS_THR, al * tb, 0.0)
    r = r + jnp.sum(wgt * cr, axis=0, keepdims=True)
    g = g + jnp.sum(wgt * cg, axis=0, keepdims=True)
    b = b + jnp.sum(wgt * cbl, axis=0, keepdims=True)
    T = T * inc[-1:, :]
    return T, r, g, b


def _tile_kernel(counts_ref, binned_ref, params_ref, out_ref):
    t = pl.program_id(0)
    cnt = counts_ref[t]
    x0 = ((t % TX) * TS).astype(jnp.float32)
    y0 = ((t // TX) * TS).astype(jnp.float32)
    lane = lax.broadcasted_iota(jnp.int32, (1, TS * TS), 1)
    px = x0 + (lane % TS).astype(jnp.float32) + 0.5
    py = y0 + (lane // TS).astype(jnp.float32) + 0.5

    T0 = jnp.ones((1, TS * TS), jnp.float32)
    z = jnp.zeros((1, TS * TS), jnp.float32)
    init = (T0, z, z, z)

    def fast_fn(_):
        nch = (cnt + (KF - 1)) // KF
        gidx = lax.broadcasted_iota(jnp.int32, (KF, 1), 0)

        def chunk(j, carry):
            pr = binned_ref[0, pl.ds(j * KF, KF), :]
            gmask = (j * KF + gidx) < cnt
            return _composite_chunk(pr, gmask, px, py, carry)

        def body(j, carry):
            return lax.cond(
                jnp.any(carry[0] > TRANS_THR), lambda c: chunk(j, c),
                lambda c: c, carry)

        return lax.fori_loop(0, nch, body, init)

    def dense_fn(_):
        def chunk(i, carry):
            pr = params_ref[pl.ds(i * KD, KD), :]
            return _composite_chunk(pr, None, px, py, carry)

        def body(i, carry):
            return lax.cond(
                jnp.any(carry[0] > TRANS_THR), lambda c: chunk(i, c),
                lambda c: c, carry)

        return lax.fori_loop(0, G // KD, body, init)

    T, r, g, b = lax.cond(cnt <= L, fast_fn, dense_fn, None)
    out_ref[0, :, :] = jnp.concatenate([r, g, b], axis=0)


def _tile_ranges(params):
    mx = params[:, 0]
    my = params[:, 1]
    a = params[:, 2]
    b = params[:, 3]
    c = params[:, 4]
    op = params[:, 8]
    lnmax = jnp.log(jnp.maximum(255.0 * op, 1e-20))
    valid = lnmax > 0.0
    lnmax = jnp.maximum(lnmax, 0.0)
    rx = jnp.sqrt(2.0 * lnmax / jnp.maximum(a - b * b / c, 1e-6)) + 0.05
    ry = jnp.sqrt(2.0 * lnmax / jnp.maximum(c - b * b / a, 1e-6)) + 0.05
    tx0 = jnp.maximum(jnp.ceil((mx - rx - (TS - 0.5)) / TS), 0.0)
    tx1 = jnp.minimum(jnp.floor((mx + rx - 0.5) / TS), TX - 1.0)
    ty0 = jnp.maximum(jnp.ceil((my - ry - (TS - 0.5)) / TS), 0.0)
    ty1 = jnp.minimum(jnp.floor((my + ry - 0.5) / TS), TY - 1.0)
    tx0 = jnp.where(valid, tx0, 1.0).astype(jnp.int32)
    tx1 = jnp.where(valid, tx1, 0.0).astype(jnp.int32)
    ty0 = jnp.where(valid, ty0, 1.0).astype(jnp.int32)
    ty1 = jnp.where(valid, ty1, 0.0).astype(jnp.int32)
    return tx0, tx1, ty0, ty1


def kernel(means2d, conics, colors, opacities, depths):
    order = jnp.argsort(lax.stop_gradient(depths))
    params = jnp.concatenate(
        [
            means2d,
            conics,
            colors,
            opacities[:, None],
            jnp.zeros((G, 7), jnp.float32),
        ],
        axis=1,
    )
    params = jnp.take(params, order, axis=0)

    tx0, tx1, ty0, ty1 = _tile_ranges(params)
    bbox = jnp.stack([tx0, tx1, ty0, ty1], axis=0)

    counts2d, binned = _sc_bin(bbox, params)
    counts = counts2d[:, :TPW].reshape(NT)

    out = pl.pallas_call(
        _tile_kernel,
        grid=(NT,),
        in_specs=[
            pl.BlockSpec(memory_space=pltpu.SMEM),
            pl.BlockSpec((1, L, 16), lambda t: (t, 0, 0)),
            pl.BlockSpec((G, 16), lambda t: (0, 0)),
        ],
        out_specs=pl.BlockSpec((1, 3, TS * TS), lambda t: (t, 0, 0)),
        out_shape=jax.ShapeDtypeStruct((NT, 3, TS * TS), jnp.float32),
        compiler_params=pltpu.CompilerParams(
            dimension_semantics=("parallel",)
        ),
    )(counts, binned, params)

    img = out.reshape(TY, TX, 3, TS, TS)
    img = jnp.transpose(img, (0, 3, 1, 4, 2)).reshape(H, W, 3)
    return img
